# Initial kernel scaffold; baseline (speedup 1.0000x reference)
import jax
import jax.numpy as jnp
from jax.experimental import pallas as pl


def kernel(x, edge_index, batch, W1, b1, W2, b2, W3, b3, g1, be1, g2, be2, g3, be3, cW1, cb1, cW2, cb2):
    raise NotImplementedError("write your pallas kernel here")


# speedup vs baseline: 11.8650x; 11.8650x over previous
import functools

import jax
import jax.numpy as jnp
from jax import lax
from jax.experimental import pallas as pl
from jax.experimental.pallas import tpu as pltpu
from jax.experimental.pallas import tpu_sc as plsc

N = 10000
E = 320000
F_IN = 128
H = 64
G = 128
NC = 2
NS = 16
NW = NC * NS
CHUNK = 128
CPT = ((E + NW * CHUNK - 1) // (NW * CHUNK) + 7) // 8 * 8
E_PAD = NW * CPT * CHUNK
N_ACC = ((N + NS - 1) // NS + 7) // 8 * 8 * NS
ROWS_PER_TILE = N_ACC // NS
DEG_W = 16


def _mesh():
    return plsc.VectorSubcoreMesh(core_axis_name="c", subcore_axis_name="s",
                                  num_cores=NC, num_subcores=NS)


def _deg_body(dst_hbm, ones_hbm, zeros_hbm, out_hbm, dst_v, ones_v, acc):
    cid = lax.axis_index("c")
    sid = lax.axis_index("s")
    g = cid * NS + sid
    pltpu.sync_copy(dst_hbm.at[pl.ds(g * CPT, CPT)], dst_v)
    pltpu.sync_copy(ones_hbm, ones_v)
    pltpu.sync_copy(zeros_hbm.at[pl.ds(sid * ROWS_PER_TILE, ROWS_PER_TILE)],
                    acc.at[pl.ds(sid * ROWS_PER_TILE, ROWS_PER_TILE)])
    plsc.subcore_barrier()

    def body(j, carry):
        pltpu.sync_copy(ones_v, acc.at[dst_v.at[j]], add=True)
        return carry

    lax.fori_loop(0, CPT, body, 0)
    plsc.subcore_barrier()
    pltpu.sync_copy(acc.at[pl.ds(sid * ROWS_PER_TILE, ROWS_PER_TILE)],
                    out_hbm.at[cid, pl.ds(sid * ROWS_PER_TILE, ROWS_PER_TILE)])


def _sc_degree(dst2d, ones, zeros16):
    return pl.kernel(
        _deg_body,
        out_type=jax.ShapeDtypeStruct((NC, N_ACC, DEG_W), jnp.float32),
        mesh=_mesh(),
        scratch_types=[
            pltpu.VMEM((CPT, CHUNK), jnp.int32),
            pltpu.VMEM((CHUNK, DEG_W), jnp.float32),
            pltpu.VMEM_SHARED((N_ACC, DEG_W), jnp.float32),
        ],
        compiler_params=pltpu.CompilerParams(use_tc_tiling_on_sc=False),
    )(dst2d, ones, zeros16)


def _scatter_body(hp_hbm, src_hbm, dst_hbm, zeros_hbm, out_hbm,
                  src_v, dst_v, rows_v, acc, sem):
    cid = lax.axis_index("c")
    sid = lax.axis_index("s")
    g = cid * NS + sid
    pltpu.sync_copy(src_hbm.at[pl.ds(g * CPT, CPT)], src_v)
    pltpu.sync_copy(dst_hbm.at[pl.ds(g * CPT, CPT)], dst_v)
    pltpu.sync_copy(zeros_hbm.at[pl.ds(sid * ROWS_PER_TILE, ROWS_PER_TILE)],
                    acc.at[pl.ds(sid * ROWS_PER_TILE, ROWS_PER_TILE)])
    plsc.subcore_barrier()

    def body(j, carry):
        pltpu.async_copy(hp_hbm.at[src_v.at[j]], rows_v, sem).wait()
        pltpu.sync_copy(rows_v, acc.at[dst_v.at[j]], add=True)
        return carry

    lax.fori_loop(0, CPT, body, 0)
    plsc.subcore_barrier()
    pltpu.sync_copy(acc.at[pl.ds(sid * ROWS_PER_TILE, ROWS_PER_TILE)],
                    out_hbm.at[cid, pl.ds(sid * ROWS_PER_TILE, ROWS_PER_TILE)])


def _sc_scatter(hp, src2d, dst2d, zeros64):
    return pl.kernel(
        _scatter_body,
        out_type=jax.ShapeDtypeStruct((NC, N_ACC, H), jnp.float32),
        mesh=_mesh(),
        scratch_types=[
            pltpu.VMEM((CPT, CHUNK), jnp.int32),
            pltpu.VMEM((CPT, CHUNK), jnp.int32),
            pltpu.VMEM((CHUNK, H), jnp.float32),
            pltpu.VMEM_SHARED((N_ACC, H), jnp.float32),
            pltpu.SemaphoreType.DMA,
        ],
        compiler_params=pltpu.CompilerParams(use_tc_tiling_on_sc=False),
    )(hp, src2d, dst2d, zeros64)


def _dinv_from_deg(deg_ref):
    deg = deg_ref[0, :N, 0:1] + deg_ref[1, :N, 0:1] + 1.0
    return lax.rsqrt(deg)


def _tc_pre_body(x_ref, w_ref, deg_ref, hp_ref):
    dinv = _dinv_from_deg(deg_ref)
    h = jnp.dot(x_ref[...], w_ref[...], preferred_element_type=jnp.float32)
    hp_ref[...] = h * dinv


def _tc_pre(x, W1, degout):
    return pl.pallas_call(
        _tc_pre_body,
        out_shape=jax.ShapeDtypeStruct((N, H), jnp.float32),
    )(x, W1, degout)


def _combine_bn_relu(acc_ref, hp_ref, deg_ref, b_ref, g_ref, be_ref):
    dinv = _dinv_from_deg(deg_ref)
    m = dinv * (acc_ref[0, :N, :] + acc_ref[1, :N, :] + hp_ref[...]) + b_ref[...]
    mean = jnp.mean(m, axis=0, keepdims=True)
    var = jnp.mean((m - mean) ** 2, axis=0, keepdims=True)
    y = g_ref[...] * (m - mean) * lax.rsqrt(var + 1e-5) + be_ref[...]
    return jnp.maximum(y, 0.0), dinv


def _tc_mid_body(acc_ref, hp_ref, deg_ref, b_ref, g_ref, be_ref, w_ref, out_ref):
    h, dinv = _combine_bn_relu(acc_ref, hp_ref, deg_ref, b_ref, g_ref, be_ref)
    out_ref[...] = jnp.dot(h, w_ref[...], preferred_element_type=jnp.float32) * dinv


def _tc_mid(acc, hp, degout, b, gam, be, Wn):
    return pl.pallas_call(
        _tc_mid_body,
        out_shape=jax.ShapeDtypeStruct((N, H), jnp.float32),
    )(acc, hp, degout, b, gam, be, Wn)


def _tc_fin_body(acc_ref, hp_ref, deg_ref, b_ref, g_ref, be_ref, batch_ref,
                 cw1_ref, cb1_ref, cw2_ref, cb2_ref, out_ref):
    h, _ = _combine_bn_relu(acc_ref, hp_ref, deg_ref, b_ref, g_ref, be_ref)
    gid = lax.broadcasted_iota(jnp.int32, (G, N), 0)
    mask = (gid == batch_ref[...]).astype(jnp.float32)
    sums = jnp.dot(mask, h, preferred_element_type=jnp.float32)
    counts = jnp.sum(mask, axis=1, keepdims=True)
    pooled = sums / jnp.maximum(counts, 1.0)
    z = jnp.maximum(
        jnp.dot(pooled, cw1_ref[...], preferred_element_type=jnp.float32)
        + cb1_ref[...], 0.0)
    out_ref[...] = (
        jnp.dot(z, cw2_ref[...], preferred_element_type=jnp.float32)
        + cb2_ref[...])


def _tc_fin(acc, hp, degout, b, gam, be, batch2d, cW1, cb1, cW2, cb2):
    return pl.pallas_call(
        _tc_fin_body,
        out_shape=jax.ShapeDtypeStruct((G, 2), jnp.float32),
    )(acc, hp, degout, b, gam, be, batch2d, cW1, cb1, cW2, cb2)


def kernel(x, edge_index, batch, W1, b1, W2, b2, W3, b3,
           g1, be1, g2, be2, g3, be3, cW1, cb1, cW2, cb2):
    src = edge_index[0].astype(jnp.int32)
    dst = edge_index[1].astype(jnp.int32)
    pad = E_PAD - E
    src2d = jnp.concatenate([src, jnp.zeros((pad,), jnp.int32)]).reshape(
        NW * CPT, CHUNK)
    dst2d = jnp.concatenate([dst, jnp.full((pad,), N, jnp.int32)]).reshape(
        NW * CPT, CHUNK)
    ones = jnp.ones((CHUNK, DEG_W), jnp.float32)
    zeros16 = jnp.zeros((N_ACC, DEG_W), jnp.float32)
    zeros64 = jnp.zeros((N_ACC, H), jnp.float32)
    batch2d = batch.astype(jnp.int32).reshape(1, N)

    degout = _sc_degree(dst2d, ones, zeros16)

    hp = _tc_pre(x, W1, degout)
    for (b, gam, be, Wn) in ((b1, g1, be1, W2), (b2, g2, be2, W3)):
        acc = _sc_scatter(hp, src2d, dst2d, zeros64)
        hp = _tc_mid(acc, hp, degout, b.reshape(1, H), gam.reshape(1, H),
                     be.reshape(1, H), Wn)
    acc = _sc_scatter(hp, src2d, dst2d, zeros64)
    return _tc_fin(acc, hp, degout, b3.reshape(1, H), g3.reshape(1, H),
                   be3.reshape(1, H), batch2d, cW1, cb1.reshape(1, H // 2),
                   cW2, cb2.reshape(1, 2))
